# Initial kernel scaffold; baseline (speedup 1.0000x reference)
import jax
import jax.numpy as jnp
from jax.experimental import pallas as pl


def kernel(Z, ref_weight):
    raise NotImplementedError("write your pallas kernel here")


# speedup vs baseline: 317.5433x; 317.5433x over previous
import functools

import jax
import jax.numpy as jnp
from jax import lax
from jax.experimental import pallas as pl
from jax.experimental.pallas import tpu as pltpu
from jax.experimental.pallas import tpu_sc as plsc

_N = 16384 * 200
_NC = 2
_NS = 16
_NW = _NC * _NS
_PER_W = _N // _NW
_LANES = 16


def _gather_sum_body(z_hbm, w_hbm, out_hbm, w_v, z_v, acc_v):
    wid = lax.axis_index("s") * _NC + lax.axis_index("c")
    pltpu.sync_copy(w_hbm, w_v)
    pltpu.sync_copy(z_hbm.at[pl.ds(wid * _PER_W, _PER_W)], z_v)

    def body(i, acc):
        idx = z_v[pl.ds(pl.multiple_of(i * _LANES, _LANES), _LANES)]
        return acc + plsc.load_gather(w_v, [idx])

    acc = lax.fori_loop(0, _PER_W // _LANES, body,
                        jnp.zeros((_LANES,), jnp.float32))
    acc_v[...] = acc
    pltpu.sync_copy(acc_v, out_hbm.at[wid])


@jax.jit
def _gather_sum(z_flat, w_pad):
    mesh = plsc.VectorSubcoreMesh(core_axis_name="c", subcore_axis_name="s")
    run = functools.partial(
        pl.kernel,
        mesh=mesh,
        compiler_params=pltpu.CompilerParams(needs_layout_passes=False),
        out_type=jax.ShapeDtypeStruct((_NW, _LANES), jnp.float32),
        scratch_types=[
            pltpu.VMEM((128,), jnp.float32),
            pltpu.VMEM((_PER_W,), jnp.int32),
            pltpu.VMEM((_LANES,), jnp.float32),
        ],
    )(_gather_sum_body)
    return run(z_flat, w_pad)


def kernel(Z, ref_weight):
    z_flat = Z.reshape(-1)
    w_pad = jnp.pad(ref_weight.reshape(-1), (0, 128 - ref_weight.shape[0]))
    partials = _gather_sum(z_flat, w_pad)
    return partials.sum()


# speedup vs baseline: 453.6412x; 1.4286x over previous
import functools

import jax
import jax.numpy as jnp
from jax import lax
from jax.experimental import pallas as pl
from jax.experimental.pallas import tpu as pltpu
from jax.experimental.pallas import tpu_sc as plsc

_N = 16384 * 200
_NC = 2
_NS = 16
_NW = _NC * _NS
_PER_W = _N // _NW
_LANES = 16


def _gather_sum_body(z_hbm, w_hbm, out_hbm, w_v, z_v, acc_v):
    wid = lax.axis_index("s") * _NC + lax.axis_index("c")
    pltpu.sync_copy(w_hbm, w_v)
    pltpu.sync_copy(z_hbm.at[pl.ds(wid * _PER_W, _PER_W)], z_v)

    zero = jnp.zeros((_LANES,), jnp.float32)

    @plsc.parallel_loop(0, _PER_W, step=4 * _LANES, unroll=4,
                       carry=(zero, zero, zero, zero))
    def body(i, accs):
        a0, a1, a2, a3 = accs
        base = pl.multiple_of(i, 4 * _LANES)
        g = [plsc.load_gather(w_v, [z_v[pl.ds(base + k * _LANES, _LANES)]])
             for k in range(4)]
        return (a0 + g[0], a1 + g[1], a2 + g[2], a3 + g[3])

    a0, a1, a2, a3 = body
    acc_v[...] = (a0 + a1) + (a2 + a3)
    pltpu.sync_copy(acc_v, out_hbm.at[wid])


@jax.jit
def _gather_sum(z_flat, w_pad):
    mesh = plsc.VectorSubcoreMesh(core_axis_name="c", subcore_axis_name="s")
    run = functools.partial(
        pl.kernel,
        mesh=mesh,
        compiler_params=pltpu.CompilerParams(needs_layout_passes=False),
        out_type=jax.ShapeDtypeStruct((_NW, _LANES), jnp.float32),
        scratch_types=[
            pltpu.VMEM((128,), jnp.float32),
            pltpu.VMEM((_PER_W,), jnp.int32),
            pltpu.VMEM((_LANES,), jnp.float32),
        ],
    )(_gather_sum_body)
    return run(z_flat, w_pad)


def kernel(Z, ref_weight):
    z_flat = Z.reshape(-1)
    w_pad = jnp.pad(ref_weight.reshape(-1), (0, 128 - ref_weight.shape[0]))
    partials = _gather_sum(z_flat, w_pad)
    return partials.sum()


# speedup vs baseline: 689.4923x; 1.5199x over previous
import functools

import jax
import jax.numpy as jnp
from jax import lax
from jax.experimental import pallas as pl
from jax.experimental.pallas import tpu as pltpu
from jax.experimental.pallas import tpu_sc as plsc

_ROWS = 16384
_COLS = 200
_NC = 2
_NS = 16
_NW = _NC * _NS
_ROWS_W = _ROWS // _NW
_LANES = 16
_FULL_GROUPS = _COLS // _LANES
_TAIL_START = _COLS - _LANES


_SLAB = 128
_N_SLABS = _ROWS_W // _SLAB


def _gather_sum_body(z_hbm, w_hbm, out_hbm, w_v, z0_v, z1_v, acc_v,
                     sem0, sem1):
    wid = lax.axis_index("s") * _NC + lax.axis_index("c")
    base = wid * _ROWS_W
    pltpu.sync_copy(w_hbm, w_v)

    zero = jnp.zeros((_LANES,), jnp.float32)
    tail_mask = lax.iota(jnp.int32, _LANES) >= (_FULL_GROUPS * _LANES - _TAIL_START)

    bufs = (z0_v, z1_v)
    sems = (sem0, sem1)
    copies = [None, None]
    copies[0] = pltpu.async_copy(z_hbm.at[pl.ds(base, _SLAB), :], z0_v, sem0)

    accs = (zero, zero, zero, zero)
    for s in range(_N_SLABS):
        if s + 1 < _N_SLABS:
            copies[(s + 1) % 2] = pltpu.async_copy(
                z_hbm.at[pl.ds(base + (s + 1) * _SLAB, _SLAB), :],
                bufs[(s + 1) % 2], sems[(s + 1) % 2])
        copies[s % 2].wait()
        z_v = bufs[s % 2]

        @plsc.parallel_loop(0, _SLAB, step=1, unroll=2, carry=accs)
        def body(r, accs):
            a = list(accs)
            for k in range(_FULL_GROUPS):
                g = plsc.load_gather(w_v, [z_v[r, pl.ds(k * _LANES, _LANES)]])
                a[k % 4] = a[k % 4] + g
            gt = plsc.load_gather(w_v, [z_v[r, pl.ds(_TAIL_START, _LANES)]])
            a[0] = a[0] + jnp.where(tail_mask, gt, 0.0)
            return tuple(a)

        accs = body

    a0, a1, a2, a3 = accs
    acc_v[...] = (a0 + a1) + (a2 + a3)
    pltpu.sync_copy(acc_v, out_hbm.at[wid])


@jax.jit
def _gather_sum(z, w_pad):
    mesh = plsc.VectorSubcoreMesh(core_axis_name="c", subcore_axis_name="s")
    run = functools.partial(
        pl.kernel,
        mesh=mesh,
        compiler_params=pltpu.CompilerParams(needs_layout_passes=False),
        out_type=jax.ShapeDtypeStruct((_NW, _LANES), jnp.float32),
        scratch_types=[
            pltpu.VMEM((128,), jnp.float32),
            pltpu.VMEM((_SLAB, _COLS), jnp.int32),
            pltpu.VMEM((_SLAB, _COLS), jnp.int32),
            pltpu.VMEM((_LANES,), jnp.float32),
            pltpu.SemaphoreType.DMA,
            pltpu.SemaphoreType.DMA,
        ],
    )(_gather_sum_body)
    return run(z, w_pad)


def kernel(Z, ref_weight):
    w_pad = jnp.pad(ref_weight.reshape(-1), (0, 128 - ref_weight.shape[0]))
    partials = _gather_sum(Z, w_pad)
    return partials.sum()


# speedup vs baseline: 820.8181x; 1.1905x over previous
import functools

import jax
import jax.numpy as jnp
from jax import lax
from jax.experimental import pallas as pl
from jax.experimental.pallas import tpu as pltpu
from jax.experimental.pallas import tpu_sc as plsc

_ROWS_T = 200
_COLS_T = 16384
_NC = 2
_NS = 16
_NW = _NC * _NS
_CW = _COLS_T // _NW
_LANES = 16
_CHUNK = 128
_N_CHUNKS = _CW // _CHUNK
_GPR = _CHUNK // _LANES


_N_TABLES = 2
_TBL = 128
_NCHAIN = _N_TABLES * _LANES


def _gather_sum_body(zt_hbm, w_hbm, out_hbm, w_v, cnt_v, z0_v, z1_v, acc_v,
                     sem0, sem1):
    wid = lax.axis_index("s") * _NC + lax.axis_index("c")
    base = wid * _CW

    zero = jnp.zeros((_LANES,), jnp.float32)
    ones = jnp.ones((_LANES,), jnp.float32)
    w_v[pl.ds(_TBL - _LANES, _LANES)] = zero
    pltpu.sync_copy(w_hbm, w_v.at[pl.ds(0, 119)])

    lane_off = lax.iota(jnp.int32, _LANES) * _TBL
    for i in range(_NCHAIN * _TBL // _LANES):
        cnt_v[pl.ds(i * _LANES, _LANES)] = zero

    bufs = (z0_v, z1_v)
    sems = (sem0, sem1)
    copies = [None, None]
    copies[0] = pltpu.async_copy(
        zt_hbm.at[:, pl.ds(base, _CHUNK)], z0_v, sem0)

    for s in range(_N_CHUNKS):
        if s + 1 < _N_CHUNKS:
            copies[(s + 1) % 2] = pltpu.async_copy(
                zt_hbm.at[:, pl.ds(base + (s + 1) * _CHUNK, _CHUNK)],
                bufs[(s + 1) % 2], sems[(s + 1) % 2])
        copies[s % 2].wait()
        z_v = bufs[s % 2]

        @plsc.parallel_loop(0, _ROWS_T, step=1, unroll=2)
        def body(r):
            for k in range(_GPR):
                idx = z_v[r, pl.ds(k * _LANES, _LANES)]
                plsc.addupdate_scatter(
                    cnt_v,
                    [idx + (lane_off + (k % _N_TABLES) * _LANES * _TBL)],
                    ones)

    acc = zero
    for g in range(_TBL // _LANES):
        c = cnt_v[pl.ds(g * _LANES, _LANES)]
        for t in range(1, _NCHAIN):
            c = c + cnt_v[pl.ds(t * _TBL + g * _LANES, _LANES)]
        acc = acc + c * w_v[pl.ds(g * _LANES, _LANES)]
    acc_v[...] = acc
    pltpu.sync_copy(acc_v, out_hbm.at[wid])


@jax.jit
def _gather_sum(zt, w_pad):
    mesh = plsc.VectorSubcoreMesh(core_axis_name="c", subcore_axis_name="s")
    run = functools.partial(
        pl.kernel,
        mesh=mesh,
        compiler_params=pltpu.CompilerParams(needs_layout_passes=False),
        out_type=jax.ShapeDtypeStruct((_NW, _LANES), jnp.float32),
        scratch_types=[
            pltpu.VMEM((_TBL,), jnp.float32),
            pltpu.VMEM((_NCHAIN * _TBL,), jnp.float32),
            pltpu.VMEM((_ROWS_T, _CHUNK), jnp.int32),
            pltpu.VMEM((_ROWS_T, _CHUNK), jnp.int32),
            pltpu.VMEM((_LANES,), jnp.float32),
            pltpu.SemaphoreType.DMA,
            pltpu.SemaphoreType.DMA,
        ],
    )(_gather_sum_body)
    return run(zt, w_pad)


def kernel(Z, ref_weight):
    partials = _gather_sum(Z.T, ref_weight.reshape(-1))
    return partials.sum()


# speedup vs baseline: 859.0370x; 1.0466x over previous
import functools

import jax
import jax.numpy as jnp
from jax import lax
from jax.experimental import pallas as pl
from jax.experimental.pallas import tpu as pltpu
from jax.experimental.pallas import tpu_sc as plsc

_ROWS_T = 200
_COLS_T = 16384
_NC = 2
_NS = 16
_NW = _NC * _NS
_CW = _COLS_T // _NW
_LANES = 16
_CHUNK = 128
_N_CHUNKS = _CW // _CHUNK
_GPR = _CHUNK // _LANES


_N_TABLES = 2
_TBL = 128
_NCHAIN = _N_TABLES * _LANES


def _gather_sum_body(zt_hbm, w_hbm, out_hbm, w_v, cnt_v, z0_v, z1_v, acc_v,
                     sem0, sem1):
    wid = lax.axis_index("s") * _NC + lax.axis_index("c")
    base = wid * _CW

    zero = jnp.zeros((_LANES,), jnp.float32)
    ones = jnp.ones((_LANES,), jnp.float32)
    w_v[pl.ds(_TBL - _LANES, _LANES)] = zero
    pltpu.sync_copy(w_hbm, w_v.at[pl.ds(0, 119)])

    for i in range(_N_TABLES * _TBL // _LANES):
        cnt_v[pl.ds(i * _LANES, _LANES)] = zero

    bufs = (z0_v, z1_v)
    sems = (sem0, sem1)
    copies = [None, None]
    copies[0] = pltpu.async_copy(
        zt_hbm.at[:, pl.ds(base, _CHUNK)], z0_v, sem0)

    for s in range(_N_CHUNKS):
        if s + 1 < _N_CHUNKS:
            copies[(s + 1) % 2] = pltpu.async_copy(
                zt_hbm.at[:, pl.ds(base + (s + 1) * _CHUNK, _CHUNK)],
                bufs[(s + 1) % 2], sems[(s + 1) % 2])
        copies[s % 2].wait()
        z_v = bufs[s % 2]

        @plsc.parallel_loop(0, _ROWS_T, step=1, unroll=4)
        def body(r):
            for k in range(_GPR):
                idx = z_v[r, pl.ds(k * _LANES, _LANES)]
                plsc.addupdate_scatter(
                    cnt_v, [idx + (k % _N_TABLES) * _TBL], ones)

    acc = zero
    for g in range(_TBL // _LANES):
        c = cnt_v[pl.ds(g * _LANES, _LANES)]
        for t in range(1, _N_TABLES):
            c = c + cnt_v[pl.ds(t * _TBL + g * _LANES, _LANES)]
        acc = acc + c * w_v[pl.ds(g * _LANES, _LANES)]
    acc_v[...] = acc
    pltpu.sync_copy(acc_v, out_hbm.at[wid])


@jax.jit
def _gather_sum(zt, w_pad):
    mesh = plsc.VectorSubcoreMesh(core_axis_name="c", subcore_axis_name="s")
    run = functools.partial(
        pl.kernel,
        mesh=mesh,
        compiler_params=pltpu.CompilerParams(needs_layout_passes=False),
        out_type=jax.ShapeDtypeStruct((_NW, _LANES), jnp.float32),
        scratch_types=[
            pltpu.VMEM((_TBL,), jnp.float32),
            pltpu.VMEM((_N_TABLES * _TBL,), jnp.float32),
            pltpu.VMEM((_ROWS_T, _CHUNK), jnp.int32),
            pltpu.VMEM((_ROWS_T, _CHUNK), jnp.int32),
            pltpu.VMEM((_LANES,), jnp.float32),
            pltpu.SemaphoreType.DMA,
            pltpu.SemaphoreType.DMA,
        ],
    )(_gather_sum_body)
    return run(zt, w_pad)


def kernel(Z, ref_weight):
    partials = _gather_sum(Z.T, ref_weight.reshape(-1))
    return partials.sum()
